# Initial kernel scaffold; baseline (speedup 1.0000x reference)
import jax
import jax.numpy as jnp
from jax.experimental import pallas as pl


def kernel(x):
    raise NotImplementedError("write your pallas kernel here")


# speedup vs baseline: 1.4709x; 1.4709x over previous
import functools

import jax
import jax.numpy as jnp
from jax import lax
from jax.experimental import pallas as pl
from jax.experimental.pallas import tpu as pltpu
from jax.experimental.pallas import tpu_sc as plsc

_SPARSE_RATIO = 0.5
_NC, _NS, _LANES = 2, 16, 16
_NW = _NC * _NS


def _wid():
    return lax.axis_index("s") * _NC + lax.axis_index("c")


def _select_body(K, score_hbm, idx_hbm, sbits_v, idxbuf_v):
    B, L = score_hbm.shape
    nv = L // _LANES
    w = _wid()

    @pl.when(w < B)
    def _():
        pltpu.sync_copy(score_hbm.at[w], sbits_v)
        lane = lax.iota(jnp.int32, _LANES)

        def count_ge(t):
            def body(i, acc):
                v = sbits_v[pl.ds(i * _LANES, _LANES)]
                return acc + jnp.where(v >= t, 1, 0)

            acc = lax.fori_loop(0, nv, body, jnp.zeros((_LANES,), jnp.int32),
                                unroll=8)
            return jnp.sum(acc)

        def step(k, prefix):
            cand = prefix | (jnp.int32(1) << (30 - k))
            return jnp.where(count_ge(cand) >= K, cand, prefix)

        thr = lax.fori_loop(0, 31, step, jnp.int32(0))
        n_eq = K - count_ge(thr + 1)

        def emit(i, carry):
            off, eq_seen = carry
            v = sbits_v[pl.ds(i * _LANES, _LANES)]
            gt = v > thr
            eq = v == thr
            eqi = jnp.where(eq, 1, 0)
            excl = plsc.cumsum(eqi) - eqi
            keep = gt | (eq & ((eq_seen + excl) < n_eq))
            plsc.store_compressed(idxbuf_v.at[pl.ds(off, _LANES)],
                                  lane + i * _LANES, mask=keep)
            return (off + jnp.sum(jnp.where(keep, 1, 0)),
                    eq_seen + jnp.sum(eqi))

        lax.fori_loop(0, nv, emit, (jnp.int32(0), jnp.int32(0)), unroll=4)
        pltpu.sync_copy(idxbuf_v.at[pl.ds(0, K)], idx_hbm.at[w])


def _sc_select(score, K):
    B, L = score.shape
    mesh = plsc.VectorSubcoreMesh(core_axis_name="c", subcore_axis_name="s")
    return pl.kernel(
        functools.partial(_select_body, K),
        out_type=jax.ShapeDtypeStruct((B, K), jnp.int32),
        mesh=mesh,
        scratch_types=[
            pltpu.VMEM((L,), jnp.int32),
            pltpu.VMEM((K + _LANES,), jnp.int32),
        ],
        compiler_params=pltpu.CompilerParams(needs_layout_passes=False),
    )(score)


_CH = 16


def _gather_body(x_hbm, idx_hbm, out_hbm, idx_v, buf0, buf1, g0, g1, s0, s1):
    B, L, C = x_hbm.shape
    K = idx_hbm.shape[1]
    per_b = _NW // B
    rows = K // per_b
    nch = rows // _CH
    w = _wid()
    b = w // per_b
    base = (w % per_b) * rows

    pltpu.sync_copy(idx_hbm.at[b, pl.ds(base, rows)], idx_v)

    bufs = (buf0, buf1)
    gsems = (g0, g1)
    ssems = (s0, s1)

    def gth(j, t):
        return pltpu.make_async_copy(
            x_hbm.at[b].at[idx_v.at[pl.ds(j * _CH, _CH)]], bufs[t], gsems[t])

    def sct(j, t):
        return pltpu.make_async_copy(
            bufs[t], out_hbm.at[b, pl.ds(base + j * _CH, _CH), :], ssems[t])

    gth(0, 0).start()
    gth(1, 1).start()

    def pair(j2, _):
        for t in range(2):
            j = j2 * 2 + t
            gth(j, t).wait()
            sct(j, t).start()
            sct(j, t).wait()
            gth(j + 2, t).start()
        return ()

    lax.fori_loop(0, nch // 2 - 1, pair, ())
    for t in range(2):
        j = nch - 2 + t
        gth(j, t).wait()
        sct(j, t).start()
        sct(j, t).wait()


def _sc_gather(x, idx):
    B, L, C = x.shape
    K = idx.shape[1]
    rows = K // (_NW // B)
    mesh = plsc.VectorSubcoreMesh(core_axis_name="c", subcore_axis_name="s")
    return pl.kernel(
        _gather_body,
        out_type=jax.ShapeDtypeStruct((B, K, C), jnp.float32),
        mesh=mesh,
        scratch_types=[
            pltpu.VMEM((rows,), jnp.int32),
            pltpu.VMEM((_CH, C), jnp.float32),
            pltpu.VMEM((_CH, C), jnp.float32),
            pltpu.SemaphoreType.DMA,
            pltpu.SemaphoreType.DMA,
            pltpu.SemaphoreType.DMA,
            pltpu.SemaphoreType.DMA,
        ],
    )(x, idx)


def kernel(x):
    B, L, C = x.shape
    K = max(1, int(L * (1.0 - _SPARSE_RATIO)))
    score = jnp.sqrt(jnp.sum(x * x, axis=-1))
    indices = _sc_select(lax.bitcast_convert_type(score, jnp.int32), K)
    x_sparse = _sc_gather(x, indices)
    return (x_sparse, indices)
